# baseline (speedup 1.0000x reference)
import functools

import jax
import jax.numpy as jnp
from jax import lax
from jax.experimental import pallas as pl
from jax.experimental.pallas import tpu as pltpu
from jax.experimental.pallas import tpu_sc as plsc

N = 10000
NP = 10048
NPK = NP // 8
E = 320000
F_IN = 128
F_HID = 16

NC = 2
NS = 16
NW = NC * NS
CHUNK = 512
EC = E // CHUNK
CPT = EC // NW
EXTRA = EC - NW * CPT
CLAST = CPT + 1
RPT = NP // NS

_mesh = plsc.VectorSubcoreMesh(core_axis_name="c", subcore_axis_name="s")


@functools.partial(
    pl.kernel,
    mesh=_mesh,
    out_type=jax.ShapeDtypeStruct((NC * NP, F_HID), jnp.float32),
    compiler_params=pltpu.CompilerParams(use_tc_tiling_on_sc=False),
    scratch_types=[
        pltpu.VMEM((CLAST * CHUNK,), jnp.int32),
        pltpu.VMEM((CHUNK, F_HID), jnp.float32),
        pltpu.VMEM_SHARED((NP, F_HID), jnp.float32),
        pltpu.SemaphoreType.DMA,
        pltpu.SemaphoreType.DMA,
        pltpu.SemaphoreType.DMA,
    ],
)
def _sc_degree(e_hbm, ones_hbm, zeros_hbm, out_hbm, dst_v, ones_v, acc,
               sem0, sem1, sem2):
    cid = lax.axis_index("c")
    sid = lax.axis_index("s")
    wid = cid * NS + sid
    nct = CPT + jnp.where(wid < EXTRA, 1, 0)
    c0 = wid * CPT + jnp.minimum(wid, EXTRA)
    h0 = pltpu.async_copy(e_hbm.at[1, pl.ds(c0 * CHUNK, CLAST * CHUNK)],
                          dst_v, sem0)
    h1 = pltpu.async_copy(ones_hbm, ones_v, sem1)
    h2 = pltpu.async_copy(zeros_hbm.at[pl.ds(sid * RPT, RPT)],
                          acc.at[pl.ds(sid * RPT, RPT)], sem2)
    h0.wait()
    h1.wait()
    h2.wait()
    plsc.subcore_barrier()

    def body(j, carry):
        pltpu.sync_copy(ones_v, acc.at[dst_v.at[pl.ds(j * CHUNK, CHUNK)]],
                        add=True)
        return carry

    lax.fori_loop(0, nct, body, 0)
    plsc.subcore_barrier()
    pltpu.sync_copy(acc.at[pl.ds(sid * RPT, RPT)],
                    out_hbm.at[pl.ds(cid * NP + sid * RPT, RPT)])


@functools.partial(
    pl.kernel,
    mesh=_mesh,
    out_type=jax.ShapeDtypeStruct((NC * NP, F_HID), jnp.float32),
    compiler_params=pltpu.CompilerParams(use_tc_tiling_on_sc=False),
    scratch_types=[
        pltpu.VMEM((CLAST * CHUNK,), jnp.int32),
        pltpu.VMEM((CLAST * CHUNK,), jnp.int32),
        pltpu.VMEM((CHUNK, F_HID), jnp.float32),
        pltpu.VMEM((CHUNK, F_HID), jnp.float32),
        pltpu.VMEM_SHARED((NP, F_HID), jnp.float32),
        pltpu.VMEM_SHARED((NP, F_HID), jnp.float32),
        pltpu.SemaphoreType.DMA,
        pltpu.SemaphoreType.DMA,
        pltpu.SemaphoreType.DMA,
        pltpu.SemaphoreType.DMA,
    ],
)
def _sc_gather_scatter(y_hbm, e_hbm, zeros_hbm, out_hbm,
                       src_v, dst_v, buf0, buf1, acc, y_s,
                       semg0, semg1, sems0, sems1):
    cid = lax.axis_index("c")
    sid = lax.axis_index("s")
    wid = cid * NS + sid
    nct = CPT + jnp.where(wid < EXTRA, 1, 0)
    c0 = wid * CPT + jnp.minimum(wid, EXTRA)
    h0 = pltpu.async_copy(e_hbm.at[0, pl.ds(c0 * CHUNK, CLAST * CHUNK)],
                          src_v, semg0)
    h1 = pltpu.async_copy(e_hbm.at[1, pl.ds(c0 * CHUNK, CLAST * CHUNK)],
                          dst_v, semg1)
    h2 = pltpu.async_copy(y_hbm.at[pl.ds(sid * RPT, RPT)],
                          y_s.at[pl.ds(sid * RPT, RPT)], sems0)
    h3 = pltpu.async_copy(zeros_hbm.at[pl.ds(sid * RPT, RPT)],
                          acc.at[pl.ds(sid * RPT, RPT)], sems1)
    h0.wait()
    h1.wait()
    h2.wait()
    h3.wait()
    plsc.subcore_barrier()

    pltpu.async_copy(y_s.at[src_v.at[pl.ds(0 * CHUNK, CHUNK)]], buf0, semg0)
    pltpu.async_copy(y_s.at[src_v.at[pl.ds(1 * CHUNK, CHUNK)]], buf1, semg1)

    def body(j0, carry):
        pltpu.make_async_copy(y_s.at[src_v.at[pl.ds(j0 * CHUNK, CHUNK)]], buf0, semg0).wait()
        pltpu.sync_copy(buf0, acc.at[dst_v.at[pl.ds(j0 * CHUNK, CHUNK)]], add=True)

        @pl.when(j0 + 2 < nct)
        def _():
            pltpu.async_copy(y_s.at[src_v.at[pl.ds((j0 + 2) * CHUNK, CHUNK)]], buf0, semg0)

        pltpu.make_async_copy(y_s.at[src_v.at[pl.ds((j0 + 1) * CHUNK, CHUNK)]], buf1, semg1).wait()
        pltpu.sync_copy(buf1, acc.at[dst_v.at[pl.ds((j0 + 1) * CHUNK, CHUNK)]], add=True)

        @pl.when(j0 + 3 < nct)
        def _():
            pltpu.async_copy(y_s.at[src_v.at[pl.ds((j0 + 3) * CHUNK, CHUNK)]], buf1, semg1)

        return carry

    lax.fori_loop(0, nct // 2, lambda p, c: body(2 * p, c), 0)

    @pl.when(nct % 2 == 1)
    def _():
        jl = nct - 1
        pltpu.make_async_copy(y_s.at[src_v.at[pl.ds(jl * CHUNK, CHUNK)]], buf0, semg0).wait()
        pltpu.sync_copy(buf0, acc.at[dst_v.at[pl.ds(jl * CHUNK, CHUNK)]], add=True)
    plsc.subcore_barrier()
    pltpu.sync_copy(acc.at[pl.ds(sid * RPT, RPT)],
                    out_hbm.at[pl.ds(cid * NP + sid * RPT, RPT)])


def _tc_pre(x_ref, w1_ref, d_ref, y_ref, dinv_ref):
    deg = d_ref[pl.ds(0, NPK), :] + d_ref[pl.ds(NPK, NPK), :] + 1.0
    dinv = lax.rsqrt(deg)
    xw = jnp.dot(x_ref[...], w1_ref[...], preferred_element_type=jnp.float32)
    y_ref[...] = dinv * xw
    dinv_ref[...] = dinv


def _tc_mid(p_ref, y1_ref, dinv_ref, b1_ref, w2_ref, y2_ref):
    dinv = dinv_ref[...]
    h = dinv * (p_ref[pl.ds(0, NPK), :] + p_ref[pl.ds(NPK, NPK), :]
                + y1_ref[...]) + b1_ref[...]
    h = jnp.maximum(h, 0.0)
    y2_ref[...] = dinv * jnp.dot(h, w2_ref[...],
                                 preferred_element_type=jnp.float32)


def _tc_post(q_ref, y2_ref, dinv_ref, b2_ref, o_ref):
    o_ref[...] = dinv_ref[...] * (q_ref[pl.ds(0, NPK), :]
                                  + q_ref[pl.ds(NPK, NPK), :]
                                  + y2_ref[...]) + b2_ref[...]


def kernel(x, edge_index, W1, b1, W2, b2):
    f32 = jnp.float32
    e3 = edge_index.astype(jnp.int32)
    x_pk = jnp.pad(x, ((0, NP - N), (0, 0))).reshape(NPK, 8 * F_IN)
    zeros = jnp.zeros((NP, F_HID), f32)
    ones_rows = jnp.ones((CHUNK, F_HID), f32)
    w1_big = jnp.kron(jnp.eye(8, dtype=f32), W1)
    w2_big = jnp.kron(jnp.eye(8, dtype=f32), W2)
    b1_big = jnp.tile(b1, 8).reshape(1, 128)
    b2_big = jnp.tile(b2, 8).reshape(1, 128)

    deg_parts = _sc_degree(e3, ones_rows, zeros)
    y1p, dinvp = pl.pallas_call(
        _tc_pre,
        out_shape=(jax.ShapeDtypeStruct((NPK, 128), f32),
                   jax.ShapeDtypeStruct((NPK, 128), f32)),
    )(x_pk, w1_big, deg_parts.reshape(2 * NPK, 128))

    p_parts = _sc_gather_scatter(y1p.reshape(NP, F_HID), e3, zeros)
    y2p = pl.pallas_call(
        _tc_mid,
        out_shape=jax.ShapeDtypeStruct((NPK, 128), f32),
    )(p_parts.reshape(2 * NPK, 128), y1p, dinvp, b1_big, w2_big)

    q_parts = _sc_gather_scatter(y2p.reshape(NP, F_HID), e3, zeros)
    outp = pl.pallas_call(
        _tc_post,
        out_shape=jax.ShapeDtypeStruct((NPK, 128), f32),
    )(q_parts.reshape(2 * NPK, 128), y2p, dinvp, b2_big)
    return outp.reshape(NP, F_HID)[:N]


# speedup vs baseline: 1.0023x; 1.0023x over previous
import functools

import jax
import jax.numpy as jnp
from jax import lax
from jax.experimental import pallas as pl
from jax.experimental.pallas import tpu as pltpu
from jax.experimental.pallas import tpu_sc as plsc

N = 10000
NP = 10048
NPK = NP // 8
E = 320000
F_IN = 128
F_HID = 16

NC = 2
NS = 16
NW = NC * NS
CHUNK = 512
EC = E // CHUNK
CPT = EC // NW
EXTRA = EC - NW * CPT
CLAST = CPT + 1
RPT = NP // NS

_mesh = plsc.VectorSubcoreMesh(core_axis_name="c", subcore_axis_name="s")


@functools.partial(
    pl.kernel,
    mesh=_mesh,
    out_type=jax.ShapeDtypeStruct((NC * NP, F_HID), jnp.float32),
    compiler_params=pltpu.CompilerParams(use_tc_tiling_on_sc=False),
    scratch_types=[
        pltpu.VMEM((CLAST * CHUNK,), jnp.int32),
        pltpu.VMEM((CHUNK, F_HID), jnp.float32),
        pltpu.VMEM_SHARED((NP, F_HID), jnp.float32),
        pltpu.SemaphoreType.DMA,
        pltpu.SemaphoreType.DMA,
        pltpu.SemaphoreType.DMA,
    ],
)
def _sc_degree(e_hbm, ones_hbm, zeros_hbm, out_hbm, dst_v, ones_v, acc,
               sem0, sem1, sem2):
    cid = lax.axis_index("c")
    sid = lax.axis_index("s")
    wid = cid * NS + sid
    nct = CPT + jnp.where(wid < EXTRA, 1, 0)
    c0 = wid * CPT + jnp.minimum(wid, EXTRA)
    h0 = pltpu.async_copy(e_hbm.at[1, pl.ds(c0 * CHUNK, CLAST * CHUNK)],
                          dst_v, sem0)
    h1 = pltpu.async_copy(ones_hbm, ones_v, sem1)
    h2 = pltpu.async_copy(zeros_hbm.at[pl.ds(sid * RPT, RPT)],
                          acc.at[pl.ds(sid * RPT, RPT)], sem2)
    h0.wait()
    h1.wait()
    h2.wait()
    plsc.subcore_barrier()

    def body(j, carry):
        pltpu.sync_copy(ones_v, acc.at[dst_v.at[pl.ds(j * CHUNK, CHUNK)]],
                        add=True)
        return carry

    lax.fori_loop(0, nct, body, 0)
    plsc.subcore_barrier()
    pltpu.sync_copy(acc.at[pl.ds(sid * RPT, RPT)],
                    out_hbm.at[pl.ds(cid * NP + sid * RPT, RPT)])


@functools.partial(
    pl.kernel,
    mesh=_mesh,
    out_type=jax.ShapeDtypeStruct((NC * NP, F_HID), jnp.float32),
    compiler_params=pltpu.CompilerParams(use_tc_tiling_on_sc=False),
    scratch_types=[
        pltpu.VMEM((CLAST * CHUNK,), jnp.int32),
        pltpu.VMEM((CLAST * CHUNK,), jnp.int32),
        pltpu.VMEM((CHUNK, F_HID), jnp.float32),
        pltpu.VMEM((CHUNK, F_HID), jnp.float32),
        pltpu.VMEM((CHUNK, F_HID), jnp.float32),
        pltpu.VMEM((CHUNK, F_HID), jnp.float32),
        pltpu.VMEM_SHARED((NP, F_HID), jnp.float32),
        pltpu.VMEM_SHARED((NP, F_HID), jnp.float32),
        pltpu.SemaphoreType.DMA,
        pltpu.SemaphoreType.DMA,
        pltpu.SemaphoreType.DMA,
        pltpu.SemaphoreType.DMA,
        pltpu.SemaphoreType.DMA,
        pltpu.SemaphoreType.DMA,
    ],
)
def _sc_gather_scatter(y_hbm, e_hbm, zeros_hbm, out_hbm,
                       src_v, dst_v, buf0, buf1, buf2, buf3, acc, y_s,
                       semg0, semg1, semg2, semg3, sems0, sems1):
    cid = lax.axis_index("c")
    sid = lax.axis_index("s")
    wid = cid * NS + sid
    nct = CPT + jnp.where(wid < EXTRA, 1, 0)
    c0 = wid * CPT + jnp.minimum(wid, EXTRA)
    h0 = pltpu.async_copy(e_hbm.at[0, pl.ds(c0 * CHUNK, CLAST * CHUNK)],
                          src_v, semg0)
    h1 = pltpu.async_copy(e_hbm.at[1, pl.ds(c0 * CHUNK, CLAST * CHUNK)],
                          dst_v, semg1)
    h2 = pltpu.async_copy(y_hbm.at[pl.ds(sid * RPT, RPT)],
                          y_s.at[pl.ds(sid * RPT, RPT)], sems0)
    h3 = pltpu.async_copy(zeros_hbm.at[pl.ds(sid * RPT, RPT)],
                          acc.at[pl.ds(sid * RPT, RPT)], sems1)
    h0.wait()
    h1.wait()
    h2.wait()
    h3.wait()
    plsc.subcore_barrier()

    bufs = (buf0, buf1, buf2, buf3)
    gsems = (semg0, semg1, semg2, semg3)
    for k in range(4):
        pltpu.async_copy(y_s.at[src_v.at[pl.ds(k * CHUNK, CHUNK)]],
                         bufs[k], gsems[k])

    def quad(q, carry):
        j0 = 4 * q
        for k in range(4):
            j = j0 + k

            @pl.when(j < nct)
            def _(j=j, k=k):
                pltpu.make_async_copy(
                    y_s.at[src_v.at[pl.ds(j * CHUNK, CHUNK)]],
                    bufs[k], gsems[k]).wait()
                pltpu.sync_copy(bufs[k],
                                acc.at[dst_v.at[pl.ds(j * CHUNK, CHUNK)]],
                                add=True)

                @pl.when(j + 4 < nct)
                def _():
                    pltpu.async_copy(
                        y_s.at[src_v.at[pl.ds((j + 4) * CHUNK, CHUNK)]],
                        bufs[k], gsems[k])
        return carry

    lax.fori_loop(0, (CLAST + 3) // 4, quad, 0)
    plsc.subcore_barrier()
    pltpu.sync_copy(acc.at[pl.ds(sid * RPT, RPT)],
                    out_hbm.at[pl.ds(cid * NP + sid * RPT, RPT)])


def _tc_pre(x_ref, w1_ref, d_ref, y_ref, dinv_ref):
    deg = d_ref[pl.ds(0, NPK), :] + d_ref[pl.ds(NPK, NPK), :] + 1.0
    dinv = lax.rsqrt(deg)
    xw = jnp.dot(x_ref[...], w1_ref[...], preferred_element_type=jnp.float32)
    y_ref[...] = dinv * xw
    dinv_ref[...] = dinv


def _tc_mid(p_ref, y1_ref, dinv_ref, b1_ref, w2_ref, y2_ref):
    dinv = dinv_ref[...]
    h = dinv * (p_ref[pl.ds(0, NPK), :] + p_ref[pl.ds(NPK, NPK), :]
                + y1_ref[...]) + b1_ref[...]
    h = jnp.maximum(h, 0.0)
    y2_ref[...] = dinv * jnp.dot(h, w2_ref[...],
                                 preferred_element_type=jnp.float32)


def _tc_post(q_ref, y2_ref, dinv_ref, b2_ref, o_ref):
    o_ref[...] = dinv_ref[...] * (q_ref[pl.ds(0, NPK), :]
                                  + q_ref[pl.ds(NPK, NPK), :]
                                  + y2_ref[...]) + b2_ref[...]


def kernel(x, edge_index, W1, b1, W2, b2):
    f32 = jnp.float32
    e3 = edge_index.astype(jnp.int32)
    x_pk = jnp.pad(x, ((0, NP - N), (0, 0))).reshape(NPK, 8 * F_IN)
    zeros = jnp.zeros((NP, F_HID), f32)
    ones_rows = jnp.ones((CHUNK, F_HID), f32)
    w1_big = jnp.kron(jnp.eye(8, dtype=f32), W1)
    w2_big = jnp.kron(jnp.eye(8, dtype=f32), W2)
    b1_big = jnp.tile(b1, 8).reshape(1, 128)
    b2_big = jnp.tile(b2, 8).reshape(1, 128)

    deg_parts = _sc_degree(e3, ones_rows, zeros)
    y1p, dinvp = pl.pallas_call(
        _tc_pre,
        out_shape=(jax.ShapeDtypeStruct((NPK, 128), f32),
                   jax.ShapeDtypeStruct((NPK, 128), f32)),
    )(x_pk, w1_big, deg_parts.reshape(2 * NPK, 128))

    p_parts = _sc_gather_scatter(y1p.reshape(NP, F_HID), e3, zeros)
    y2p = pl.pallas_call(
        _tc_mid,
        out_shape=jax.ShapeDtypeStruct((NPK, 128), f32),
    )(p_parts.reshape(2 * NPK, 128), y1p, dinvp, b1_big, w2_big)

    q_parts = _sc_gather_scatter(y2p.reshape(NP, F_HID), e3, zeros)
    outp = pl.pallas_call(
        _tc_post,
        out_shape=jax.ShapeDtypeStruct((NPK, 128), f32),
    )(q_parts.reshape(2 * NPK, 128), y2p, dinvp, b2_big)
    return outp.reshape(NP, F_HID)[:N]
